# baseline (speedup 1.0000x reference)
import functools

import jax
import jax.numpy as jnp
from jax import lax
from jax.experimental import pallas as pl
from jax.experimental.pallas import tpu as pltpu
from jax.experimental.pallas import tpu_sc as plsc

B = 4096
N_DEEP = 26
DEEP_VOCAB = 100000
DEEP_DIM = 100
ITEM_VOCAB = 1000000
ITEM_DIM = 64
NUM_WIDE = 26
CROSS = 128

SUPER = 2 * DEEP_DIM
SROWS = N_DEEP * DEEP_VOCAB // 2

NUM_CORES = 2
NUM_SUBCORES = 16
NW = NUM_CORES * NUM_SUBCORES
BPW = B // NW


@functools.lru_cache(maxsize=1)
def _sc_gather_build():
    mesh = plsc.VectorSubcoreMesh(core_axis_name="c", subcore_axis_name="s")

    @functools.partial(
        pl.kernel,
        mesh=mesh,
        out_type=(
            jax.ShapeDtypeStruct((N_DEEP, B, SUPER), jnp.float32),
            jax.ShapeDtypeStruct((B, ITEM_DIM), jnp.float32),
        ),
        scratch_types=[
            pltpu.VMEM((BPW,), jnp.int32),
            pltpu.VMEM((BPW,), jnp.int32),
            pltpu.VMEM((BPW, SUPER), jnp.float32),
            pltpu.VMEM((BPW, ITEM_DIM), jnp.float32),
            pltpu.SemaphoreType.DMA,
        ],
        compiler_params=pltpu.CompilerParams(use_tc_tiling_on_sc=False),
    )
    def sc_gather(
        deep2_hbm,
        half_idx_hbm,
        item_hbm,
        dev_idx_hbm,
        deep_out_hbm,
        dev_out_hbm,
        dev_idx_v,
        cur_idx_v,
        rows_v,
        item_rows_v,
        gsem,
    ):
        wid = lax.axis_index("s") * NUM_CORES + lax.axis_index("c")
        base = wid * BPW

        pltpu.sync_copy(dev_idx_hbm.at[pl.ds(base, BPW)], dev_idx_v)
        pltpu.async_copy(item_hbm.at[dev_idx_v], item_rows_v, gsem).wait()
        pltpu.sync_copy(item_rows_v, dev_out_hbm.at[pl.ds(base, BPW)])

        def body(t, _):
            pltpu.sync_copy(half_idx_hbm.at[pl.ds(t * B + base, BPW)], cur_idx_v)
            pltpu.async_copy(deep2_hbm.at[cur_idx_v], rows_v, gsem).wait()
            pltpu.sync_copy(rows_v, deep_out_hbm.at[t, pl.ds(base, BPW)])
            return _

        lax.fori_loop(0, N_DEEP, body, None)

    return sc_gather


def _tc_body(g_ref, par_ref, dev_ref, wide_ref, wd_ref, wdev_ref, wwide_ref,
             b_ref, out_ref):
    i = pl.program_id(1)

    @pl.when(i == 0)
    def _init():
        dev = jnp.nan_to_num(dev_ref[...])
        wblk = wide_ref[...]
        mean = jnp.mean(wblk, axis=0, keepdims=True)
        var = jnp.mean(jnp.square(wblk - mean), axis=0, keepdims=True)
        wn = (wblk - mean) * lax.rsqrt(var + 1e-5)
        wide_part = lax.dot_general(
            wn, wwide_ref[...], (((0,), (0,)), ((), ())),
            preferred_element_type=jnp.float32,
            precision=lax.Precision.HIGHEST,
        )
        dev_part = jnp.dot(
            dev, wdev_ref[...],
            preferred_element_type=jnp.float32,
            precision=lax.Precision.HIGHEST,
        )
        out_ref[...] = dev_part + wide_part + b_ref[...]

    g200 = g_ref[0]
    p = par_ref[0]
    g = jnp.where(p > 0.5, g200[:, DEEP_DIM:SUPER], g200[:, :DEEP_DIM])
    out_ref[...] += jnp.dot(
        g, wd_ref[0],
        preferred_element_type=jnp.float32,
        precision=lax.Precision.HIGHEST,
    )


def _tc_matmul(gathered, par_t, dev, wide_in, wd, wdev, wwide, b2):
    BT = 512
    grid = (B // BT, N_DEEP)
    return pl.pallas_call(
        _tc_body,
        grid=grid,
        in_specs=[
            pl.BlockSpec((1, BT, SUPER), lambda bb, i: (i, bb, 0)),
            pl.BlockSpec((1, BT, 1), lambda bb, i: (i, bb, 0)),
            pl.BlockSpec((BT, ITEM_DIM), lambda bb, i: (bb, 0)),
            pl.BlockSpec((NUM_WIDE, BT), lambda bb, i: (0, bb)),
            pl.BlockSpec((1, DEEP_DIM, CROSS), lambda bb, i: (i, 0, 0)),
            pl.BlockSpec((ITEM_DIM, CROSS), lambda bb, i: (0, 0)),
            pl.BlockSpec((NUM_WIDE, CROSS), lambda bb, i: (0, 0)),
            pl.BlockSpec((1, CROSS), lambda bb, i: (0, 0)),
        ],
        out_specs=pl.BlockSpec((BT, CROSS), lambda bb, i: (bb, 0)),
        out_shape=jax.ShapeDtypeStruct((B, CROSS), jnp.float32),
        compiler_params=pltpu.CompilerParams(
            dimension_semantics=("parallel", "arbitrary"),
        ),
    )(gathered, par_t, dev, wide_in, wd, wdev, wwide, b2)


def kernel(deep_in, wide_in, device_in, deep_tables, item_table, ln_gamma, ln_beta, W, b):
    deep_in = deep_in.astype(jnp.int32)
    offs = (jnp.arange(N_DEEP, dtype=jnp.int32) * DEEP_VOCAB)[:, None]
    flat_idx = (deep_in + offs).reshape(N_DEEP * B)
    half_idx = flat_idx >> 1
    par_t = (deep_in & 1).astype(jnp.float32)[:, :, None]
    deep2 = deep_tables.reshape(SROWS, SUPER)

    gathered, dev = _sc_gather_build()(
        deep2, half_idx, item_table, device_in.astype(jnp.int32)
    )

    wd = W[: N_DEEP * DEEP_DIM].reshape(N_DEEP, DEEP_DIM, CROSS)
    wdev = W[N_DEEP * DEEP_DIM : N_DEEP * DEEP_DIM + ITEM_DIM]
    w_wide_raw = W[N_DEEP * DEEP_DIM + ITEM_DIM :]
    wwide = ln_gamma[:, None] * w_wide_raw
    b2 = (b + ln_beta @ w_wide_raw).reshape(1, CROSS)

    return _tc_matmul(gathered, par_t, dev, wide_in, wd, wdev, wwide, b2)


# speedup vs baseline: 1.4260x; 1.4260x over previous
import functools

import jax
import jax.numpy as jnp
from jax import lax
from jax.experimental import pallas as pl
from jax.experimental.pallas import tpu as pltpu
from jax.experimental.pallas import tpu_sc as plsc

B = 4096
N_DEEP = 26
DEEP_VOCAB = 100000
DEEP_DIM = 100
ITEM_VOCAB = 1000000
ITEM_DIM = 64
NUM_WIDE = 26
CROSS = 128

NUM_CORES = 2
NUM_SUBCORES = 16
NW = NUM_CORES * NUM_SUBCORES
BPW = B // NW

VC = 512


def _project_deep_body(t_ref, w_ref, p_ref):
    tb = t_ref[0].astype(jnp.bfloat16)
    w = w_ref[0].astype(jnp.bfloat16)
    p_ref[0] = lax.dot_general(
        tb, w, (((0,), (0,)), ((), ())),
        preferred_element_type=jnp.float32,
    )


def _project_deep(deep_t, wd):
    nvb = (DEEP_VOCAB + VC - 1) // VC
    return pl.pallas_call(
        _project_deep_body,
        grid=(N_DEEP, nvb),
        in_specs=[
            pl.BlockSpec((1, DEEP_DIM, VC), lambda i, v: (i, 0, v)),
            pl.BlockSpec((1, DEEP_DIM, CROSS), lambda i, v: (i, 0, 0)),
        ],
        out_specs=pl.BlockSpec((1, VC, CROSS), lambda i, v: (i, v, 0)),
        out_shape=jax.ShapeDtypeStruct((N_DEEP, DEEP_VOCAB, CROSS), jnp.float32),
        compiler_params=pltpu.CompilerParams(
            dimension_semantics=("parallel", "parallel"),
        ),
    )(deep_t, wd)


def _project_item_body(t_ref, w_ref, p_ref):
    tb = t_ref[...].astype(jnp.bfloat16)
    w = w_ref[...].astype(jnp.bfloat16)
    p_ref[...] = lax.dot_general(
        tb, w, (((0,), (0,)), ((), ())),
        preferred_element_type=jnp.float32,
    )


def _project_item(item_t, wdev):
    nvb = (ITEM_VOCAB + VC - 1) // VC
    return pl.pallas_call(
        _project_item_body,
        grid=(nvb,),
        in_specs=[
            pl.BlockSpec((ITEM_DIM, VC), lambda v: (0, v)),
            pl.BlockSpec((ITEM_DIM, CROSS), lambda v: (0, 0)),
        ],
        out_specs=pl.BlockSpec((VC, CROSS), lambda v: (v, 0)),
        out_shape=jax.ShapeDtypeStruct((ITEM_VOCAB, CROSS), jnp.float32),
        compiler_params=pltpu.CompilerParams(
            dimension_semantics=("parallel",),
        ),
    )(item_t, wdev)


@functools.lru_cache(maxsize=1)
def _sc_gather_build():
    mesh = plsc.VectorSubcoreMesh(core_axis_name="c", subcore_axis_name="s")

    @functools.partial(
        pl.kernel,
        mesh=mesh,
        out_type=(
            jax.ShapeDtypeStruct((N_DEEP, B, CROSS), jnp.float32),
            jax.ShapeDtypeStruct((B, CROSS), jnp.float32),
        ),
        scratch_types=[
            pltpu.VMEM((BPW,), jnp.int32),
            pltpu.VMEM((BPW,), jnp.int32),
            pltpu.VMEM((BPW, CROSS), jnp.float32),
            pltpu.VMEM((BPW, CROSS), jnp.float32),
            pltpu.SemaphoreType.DMA,
        ],
    )
    def sc_gather(
        p_deep_hbm,
        deep_idx_hbm,
        p_item_hbm,
        dev_idx_hbm,
        deep_out_hbm,
        dev_out_hbm,
        dev_idx_v,
        cur_idx_v,
        rows_v,
        item_rows_v,
        gsem,
    ):
        wid = lax.axis_index("s") * NUM_CORES + lax.axis_index("c")
        base = wid * BPW

        pltpu.sync_copy(dev_idx_hbm.at[pl.ds(base, BPW)], dev_idx_v)
        pltpu.async_copy(p_item_hbm.at[dev_idx_v], item_rows_v, gsem).wait()
        pltpu.sync_copy(item_rows_v, dev_out_hbm.at[pl.ds(base, BPW)])

        def body(t, _):
            pltpu.sync_copy(deep_idx_hbm.at[pl.ds(t * B + base, BPW)], cur_idx_v)
            pltpu.async_copy(p_deep_hbm.at[cur_idx_v], rows_v, gsem).wait()
            pltpu.sync_copy(rows_v, deep_out_hbm.at[t, pl.ds(base, BPW)])
            return _

        lax.fori_loop(0, N_DEEP, body, None)

    return sc_gather


def _combine_body(g_ref, dev_ref, wide_ref, wwide_ref, b_ref, out_ref):
    acc = jnp.sum(g_ref[...], axis=0) + dev_ref[...]
    wblk = wide_ref[...]
    mean = jnp.mean(wblk, axis=0, keepdims=True)
    var = jnp.mean(jnp.square(wblk - mean), axis=0, keepdims=True)
    wn = (wblk - mean) * lax.rsqrt(var + 1e-5)
    wide_part = lax.dot_general(
        wn, wwide_ref[...], (((0,), (0,)), ((), ())),
        preferred_element_type=jnp.float32,
        precision=lax.Precision.HIGHEST,
    )
    out_ref[...] = acc + wide_part + b_ref[...]


def _combine(gathered, dev, wide_in, wwide, b2):
    BT = 512
    return pl.pallas_call(
        _combine_body,
        grid=(B // BT,),
        in_specs=[
            pl.BlockSpec((N_DEEP, BT, CROSS), lambda bb: (0, bb, 0)),
            pl.BlockSpec((BT, CROSS), lambda bb: (bb, 0)),
            pl.BlockSpec((NUM_WIDE, BT), lambda bb: (0, bb)),
            pl.BlockSpec((NUM_WIDE, CROSS), lambda bb: (0, 0)),
            pl.BlockSpec((1, CROSS), lambda bb: (0, 0)),
        ],
        out_specs=pl.BlockSpec((BT, CROSS), lambda bb: (bb, 0)),
        out_shape=jax.ShapeDtypeStruct((B, CROSS), jnp.float32),
        compiler_params=pltpu.CompilerParams(
            dimension_semantics=("parallel",),
        ),
    )(gathered, dev, wide_in, wwide, b2)


def kernel(deep_in, wide_in, device_in, deep_tables, item_table, ln_gamma, ln_beta, W, b):
    deep_in = deep_in.astype(jnp.int32)
    deep_t = jnp.transpose(deep_tables, (0, 2, 1))
    item_t = jnp.transpose(item_table)

    wd = W[: N_DEEP * DEEP_DIM].reshape(N_DEEP, DEEP_DIM, CROSS)
    wdev = W[N_DEEP * DEEP_DIM : N_DEEP * DEEP_DIM + ITEM_DIM]
    w_wide_raw = W[N_DEEP * DEEP_DIM + ITEM_DIM :]
    wwide = ln_gamma[:, None] * w_wide_raw
    b2 = (b + ln_beta @ w_wide_raw).reshape(1, CROSS)

    p_deep = _project_deep(deep_t, wd).reshape(N_DEEP * DEEP_VOCAB, CROSS)
    p_item = _project_item(item_t, wdev)

    offs = (jnp.arange(N_DEEP, dtype=jnp.int32) * DEEP_VOCAB)[:, None]
    flat_idx = (deep_in + offs).reshape(N_DEEP * B)
    gathered, dev = _sc_gather_build()(
        p_deep, flat_idx, p_item, device_in.astype(jnp.int32)
    )

    return _combine(gathered, dev, wide_in, wwide, b2)


# speedup vs baseline: 3.5819x; 2.5119x over previous
import functools

import jax
import jax.numpy as jnp
from jax import lax
from jax.experimental import pallas as pl
from jax.experimental.pallas import tpu as pltpu
from jax.experimental.pallas import tpu_sc as plsc

B = 4096
N_DEEP = 26
DEEP_VOCAB = 100000
DEEP_DIM = 100
ITEM_VOCAB = 1000000
ITEM_DIM = 64
NUM_WIDE = 26
CROSS = 128

NUM_CORES = 2
NUM_SUBCORES = 16
NW = NUM_CORES * NUM_SUBCORES
BPW = B // NW

VC = 2048


def _project_deep_body(t_ref, w_ref, p_ref):
    tb = t_ref[0].astype(jnp.bfloat16)
    w = w_ref[0].astype(jnp.bfloat16)
    p_ref[0] = lax.dot_general(
        tb, w, (((0,), (0,)), ((), ())),
        preferred_element_type=jnp.float32,
    )


def _project_deep(deep_t, wd):
    nvb = (DEEP_VOCAB + VC - 1) // VC
    return pl.pallas_call(
        _project_deep_body,
        grid=(N_DEEP, nvb),
        in_specs=[
            pl.BlockSpec((1, DEEP_DIM, VC), lambda i, v: (i, 0, v)),
            pl.BlockSpec((1, DEEP_DIM, CROSS), lambda i, v: (i, 0, 0)),
        ],
        out_specs=pl.BlockSpec((1, VC, CROSS), lambda i, v: (i, v, 0)),
        out_shape=jax.ShapeDtypeStruct((N_DEEP, DEEP_VOCAB, CROSS), jnp.float32),
        compiler_params=pltpu.CompilerParams(
            dimension_semantics=("parallel", "parallel"),
        ),
    )(deep_t, wd)


def _project_item_body(t_ref, w_ref, p_ref):
    tb = t_ref[...].astype(jnp.bfloat16)
    w = w_ref[...].astype(jnp.bfloat16)
    p_ref[...] = lax.dot_general(
        tb, w, (((0,), (0,)), ((), ())),
        preferred_element_type=jnp.float32,
    )


def _project_item(item_t, wdev):
    nvb = (ITEM_VOCAB + VC - 1) // VC
    return pl.pallas_call(
        _project_item_body,
        grid=(nvb,),
        in_specs=[
            pl.BlockSpec((ITEM_DIM, VC), lambda v: (0, v)),
            pl.BlockSpec((ITEM_DIM, CROSS), lambda v: (0, 0)),
        ],
        out_specs=pl.BlockSpec((VC, CROSS), lambda v: (v, 0)),
        out_shape=jax.ShapeDtypeStruct((ITEM_VOCAB, CROSS), jnp.float32),
        compiler_params=pltpu.CompilerParams(
            dimension_semantics=("parallel",),
        ),
    )(item_t, wdev)


@functools.lru_cache(maxsize=1)
def _sc_gather_build():
    mesh = plsc.VectorSubcoreMesh(core_axis_name="c", subcore_axis_name="s")

    @functools.partial(
        pl.kernel,
        mesh=mesh,
        out_type=(
            jax.ShapeDtypeStruct((N_DEEP, B, CROSS), jnp.float32),
            jax.ShapeDtypeStruct((B, CROSS), jnp.float32),
        ),
        scratch_types=[
            pltpu.VMEM((BPW,), jnp.int32),
            pltpu.VMEM((BPW,), jnp.int32),
            pltpu.VMEM((BPW, CROSS), jnp.float32),
            pltpu.VMEM((BPW, CROSS), jnp.float32),
            pltpu.SemaphoreType.DMA,
        ],
    )
    def sc_gather(
        p_deep_hbm,
        deep_idx_hbm,
        p_item_hbm,
        dev_idx_hbm,
        deep_out_hbm,
        dev_out_hbm,
        dev_idx_v,
        cur_idx_v,
        rows_v,
        item_rows_v,
        gsem,
    ):
        wid = lax.axis_index("s") * NUM_CORES + lax.axis_index("c")
        base = wid * BPW

        pltpu.sync_copy(dev_idx_hbm.at[pl.ds(base, BPW)], dev_idx_v)
        pltpu.async_copy(p_item_hbm.at[dev_idx_v], item_rows_v, gsem).wait()
        pltpu.sync_copy(item_rows_v, dev_out_hbm.at[pl.ds(base, BPW)])

        def body(t, _):
            pltpu.sync_copy(deep_idx_hbm.at[pl.ds(t * B + base, BPW)], cur_idx_v)
            pltpu.async_copy(p_deep_hbm.at[cur_idx_v], rows_v, gsem).wait()
            pltpu.sync_copy(rows_v, deep_out_hbm.at[t, pl.ds(base, BPW)])
            return _

        lax.fori_loop(0, N_DEEP, body, None)

    return sc_gather


def _combine_body(g_ref, dev_ref, wide_ref, wwide_ref, b_ref, out_ref):
    acc = jnp.sum(g_ref[...], axis=0) + dev_ref[...]
    wblk = wide_ref[...]
    mean = jnp.mean(wblk, axis=0, keepdims=True)
    var = jnp.mean(jnp.square(wblk - mean), axis=0, keepdims=True)
    wn = (wblk - mean) * lax.rsqrt(var + 1e-5)
    wide_part = lax.dot_general(
        wn, wwide_ref[...], (((0,), (0,)), ((), ())),
        preferred_element_type=jnp.float32,
        precision=lax.Precision.HIGHEST,
    )
    out_ref[...] = acc + wide_part + b_ref[...]


def _combine(gathered, dev, wide_in, wwide, b2):
    BT = 512
    return pl.pallas_call(
        _combine_body,
        grid=(B // BT,),
        in_specs=[
            pl.BlockSpec((N_DEEP, BT, CROSS), lambda bb: (0, bb, 0)),
            pl.BlockSpec((BT, CROSS), lambda bb: (bb, 0)),
            pl.BlockSpec((NUM_WIDE, BT), lambda bb: (0, bb)),
            pl.BlockSpec((NUM_WIDE, CROSS), lambda bb: (0, 0)),
            pl.BlockSpec((1, CROSS), lambda bb: (0, 0)),
        ],
        out_specs=pl.BlockSpec((BT, CROSS), lambda bb: (bb, 0)),
        out_shape=jax.ShapeDtypeStruct((B, CROSS), jnp.float32),
        compiler_params=pltpu.CompilerParams(
            dimension_semantics=("parallel",),
        ),
    )(gathered, dev, wide_in, wwide, b2)


def kernel(deep_in, wide_in, device_in, deep_tables, item_table, ln_gamma, ln_beta, W, b):
    deep_in = deep_in.astype(jnp.int32)
    deep_t = jnp.transpose(deep_tables, (0, 2, 1))
    item_t = jnp.transpose(item_table)

    wd = W[: N_DEEP * DEEP_DIM].reshape(N_DEEP, DEEP_DIM, CROSS)
    wdev = W[N_DEEP * DEEP_DIM : N_DEEP * DEEP_DIM + ITEM_DIM]
    w_wide_raw = W[N_DEEP * DEEP_DIM + ITEM_DIM :]
    wwide = ln_gamma[:, None] * w_wide_raw
    b2 = (b + ln_beta @ w_wide_raw).reshape(1, CROSS)

    p_deep = _project_deep(deep_t, wd).reshape(N_DEEP * DEEP_VOCAB, CROSS)
    p_item = _project_item(item_t, wdev)

    offs = (jnp.arange(N_DEEP, dtype=jnp.int32) * DEEP_VOCAB)[:, None]
    flat_idx = (deep_in + offs).reshape(N_DEEP * B)
    gathered, dev = _sc_gather_build()(
        p_deep, flat_idx, p_item, device_in.astype(jnp.int32)
    )

    return _combine(gathered, dev, wide_in, wwide, b2)


# speedup vs baseline: 4.8661x; 1.3585x over previous
import functools

import jax
import jax.numpy as jnp
from jax import lax
from jax.experimental import pallas as pl
from jax.experimental.pallas import tpu as pltpu
from jax.experimental.pallas import tpu_sc as plsc

B = 4096
N_DEEP = 26
DEEP_VOCAB = 100000
DEEP_DIM = 100
ITEM_VOCAB = 1000000
ITEM_DIM = 64
NUM_WIDE = 26
CROSS = 128

NUM_CORES = 2
NUM_SUBCORES = 16
NW = NUM_CORES * NUM_SUBCORES
BPW = B // NW

VC = 4096


def _project_deep_body(t_ref, w_ref, p_ref):
    tb = t_ref[0].astype(jnp.bfloat16)
    w = w_ref[0].astype(jnp.bfloat16)
    p_ref[0] = lax.dot_general(
        tb, w, (((0,), (0,)), ((), ())),
        preferred_element_type=jnp.float32,
    )


def _project_deep(deep_t, wd):
    nvb = (DEEP_VOCAB + VC - 1) // VC
    return pl.pallas_call(
        _project_deep_body,
        grid=(N_DEEP, nvb),
        in_specs=[
            pl.BlockSpec((1, DEEP_DIM, VC), lambda i, v: (i, 0, v)),
            pl.BlockSpec((1, DEEP_DIM, CROSS), lambda i, v: (i, 0, 0)),
        ],
        out_specs=pl.BlockSpec((1, VC, CROSS), lambda i, v: (i, v, 0)),
        out_shape=jax.ShapeDtypeStruct((N_DEEP, DEEP_VOCAB, CROSS), jnp.float32),
        compiler_params=pltpu.CompilerParams(
            dimension_semantics=("parallel", "parallel"),
        ),
    )(deep_t, wd)


def _project_item_body(t_ref, w_ref, p_ref):
    tb = t_ref[...].astype(jnp.bfloat16)
    w = w_ref[...].astype(jnp.bfloat16)
    p_ref[...] = lax.dot_general(
        tb, w, (((0,), (0,)), ((), ())),
        preferred_element_type=jnp.float32,
    )


def _project_item(item_t, wdev):
    nvb = (ITEM_VOCAB + VC - 1) // VC
    return pl.pallas_call(
        _project_item_body,
        grid=(nvb,),
        in_specs=[
            pl.BlockSpec((ITEM_DIM, VC), lambda v: (0, v)),
            pl.BlockSpec((ITEM_DIM, CROSS), lambda v: (0, 0)),
        ],
        out_specs=pl.BlockSpec((VC, CROSS), lambda v: (v, 0)),
        out_shape=jax.ShapeDtypeStruct((ITEM_VOCAB, CROSS), jnp.float32),
        compiler_params=pltpu.CompilerParams(
            dimension_semantics=("parallel",),
        ),
    )(item_t, wdev)


@functools.lru_cache(maxsize=1)
def _sc_gather_build():
    mesh = plsc.VectorSubcoreMesh(core_axis_name="c", subcore_axis_name="s")

    @functools.partial(
        pl.kernel,
        mesh=mesh,
        out_type=(
            jax.ShapeDtypeStruct((N_DEEP, B, CROSS), jnp.float32),
            jax.ShapeDtypeStruct((B, CROSS), jnp.float32),
        ),
        scratch_types=[
            pltpu.VMEM((BPW,), jnp.int32),
            pltpu.VMEM((BPW,), jnp.int32),
            pltpu.VMEM((BPW, CROSS), jnp.float32),
            pltpu.VMEM((BPW, CROSS), jnp.float32),
            pltpu.SemaphoreType.DMA,
        ],
    )
    def sc_gather(
        p_deep_hbm,
        deep_idx_hbm,
        p_item_hbm,
        dev_idx_hbm,
        deep_out_hbm,
        dev_out_hbm,
        dev_idx_v,
        cur_idx_v,
        rows_v,
        item_rows_v,
        gsem,
    ):
        wid = lax.axis_index("s") * NUM_CORES + lax.axis_index("c")
        base = wid * BPW

        pltpu.sync_copy(dev_idx_hbm.at[pl.ds(base, BPW)], dev_idx_v)
        pltpu.async_copy(p_item_hbm.at[dev_idx_v], item_rows_v, gsem).wait()
        pltpu.sync_copy(item_rows_v, dev_out_hbm.at[pl.ds(base, BPW)])

        def body(t, _):
            pltpu.sync_copy(deep_idx_hbm.at[pl.ds(t * B + base, BPW)], cur_idx_v)
            pltpu.async_copy(p_deep_hbm.at[cur_idx_v], rows_v, gsem).wait()
            pltpu.sync_copy(rows_v, deep_out_hbm.at[t, pl.ds(base, BPW)])
            return _

        lax.fori_loop(0, N_DEEP, body, None)

    return sc_gather


def _combine_body(g_ref, dev_ref, wide_ref, wwide_ref, b_ref, out_ref):
    acc = jnp.sum(g_ref[...], axis=0) + dev_ref[...]
    wblk = wide_ref[...]
    mean = jnp.mean(wblk, axis=0, keepdims=True)
    var = jnp.mean(jnp.square(wblk - mean), axis=0, keepdims=True)
    wn = (wblk - mean) * lax.rsqrt(var + 1e-5)
    wide_part = lax.dot_general(
        wn, wwide_ref[...], (((0,), (0,)), ((), ())),
        preferred_element_type=jnp.float32,
        precision=lax.Precision.HIGHEST,
    )
    out_ref[...] = acc + wide_part + b_ref[...]


def _combine(gathered, dev, wide_in, wwide, b2):
    BT = 512
    return pl.pallas_call(
        _combine_body,
        grid=(B // BT,),
        in_specs=[
            pl.BlockSpec((N_DEEP, BT, CROSS), lambda bb: (0, bb, 0)),
            pl.BlockSpec((BT, CROSS), lambda bb: (bb, 0)),
            pl.BlockSpec((NUM_WIDE, BT), lambda bb: (0, bb)),
            pl.BlockSpec((NUM_WIDE, CROSS), lambda bb: (0, 0)),
            pl.BlockSpec((1, CROSS), lambda bb: (0, 0)),
        ],
        out_specs=pl.BlockSpec((BT, CROSS), lambda bb: (bb, 0)),
        out_shape=jax.ShapeDtypeStruct((B, CROSS), jnp.float32),
        compiler_params=pltpu.CompilerParams(
            dimension_semantics=("parallel",),
        ),
    )(gathered, dev, wide_in, wwide, b2)


def kernel(deep_in, wide_in, device_in, deep_tables, item_table, ln_gamma, ln_beta, W, b):
    deep_in = deep_in.astype(jnp.int32)
    deep_t = jnp.transpose(deep_tables, (0, 2, 1))
    item_t = jnp.transpose(item_table)

    wd = W[: N_DEEP * DEEP_DIM].reshape(N_DEEP, DEEP_DIM, CROSS)
    wdev = W[N_DEEP * DEEP_DIM : N_DEEP * DEEP_DIM + ITEM_DIM]
    w_wide_raw = W[N_DEEP * DEEP_DIM + ITEM_DIM :]
    wwide = ln_gamma[:, None] * w_wide_raw
    b2 = (b + ln_beta @ w_wide_raw).reshape(1, CROSS)

    p_deep = _project_deep(deep_t, wd).reshape(N_DEEP * DEEP_VOCAB, CROSS)
    p_item = _project_item(item_t, wdev)

    offs = (jnp.arange(N_DEEP, dtype=jnp.int32) * DEEP_VOCAB)[:, None]
    flat_idx = (deep_in + offs).reshape(N_DEEP * B)
    gathered, dev = _sc_gather_build()(
        p_deep, flat_idx, p_item, device_in.astype(jnp.int32)
    )

    return _combine(gathered, dev, wide_in, wwide, b2)


# speedup vs baseline: 5.8449x; 1.2011x over previous
import functools

import jax
import jax.numpy as jnp
from jax import lax
from jax.experimental import pallas as pl
from jax.experimental.pallas import tpu as pltpu
from jax.experimental.pallas import tpu_sc as plsc

B = 4096
N_DEEP = 26
DEEP_VOCAB = 100000
DEEP_DIM = 100
ITEM_VOCAB = 1000000
ITEM_DIM = 64
NUM_WIDE = 26
CROSS = 128

NUM_CORES = 2
NUM_SUBCORES = 16
NW = NUM_CORES * NUM_SUBCORES
BPW = B // NW

VC = 8192


def _project_deep_body(t_ref, w_ref, p_ref):
    tb = t_ref[0].astype(jnp.bfloat16)
    w = w_ref[0].astype(jnp.bfloat16)
    p_ref[0] = lax.dot_general(
        tb, w, (((0,), (0,)), ((), ())),
        preferred_element_type=jnp.float32,
    )


def _project_deep(deep_t, wd):
    nvb = (DEEP_VOCAB + VC - 1) // VC
    return pl.pallas_call(
        _project_deep_body,
        grid=(N_DEEP, nvb),
        in_specs=[
            pl.BlockSpec((1, DEEP_DIM, VC), lambda i, v: (i, 0, v)),
            pl.BlockSpec((1, DEEP_DIM, CROSS), lambda i, v: (i, 0, 0)),
        ],
        out_specs=pl.BlockSpec((1, VC, CROSS), lambda i, v: (i, v, 0)),
        out_shape=jax.ShapeDtypeStruct((N_DEEP, DEEP_VOCAB, CROSS), jnp.float32),
        compiler_params=pltpu.CompilerParams(
            dimension_semantics=("parallel", "parallel"),
        ),
    )(deep_t, wd)


def _project_item_body(t_ref, w_ref, p_ref):
    tb = t_ref[...].astype(jnp.bfloat16)
    w = w_ref[...].astype(jnp.bfloat16)
    p_ref[...] = lax.dot_general(
        tb, w, (((0,), (0,)), ((), ())),
        preferred_element_type=jnp.float32,
    )


def _project_item(item_t, wdev):
    nvb = (ITEM_VOCAB + VC - 1) // VC
    return pl.pallas_call(
        _project_item_body,
        grid=(nvb,),
        in_specs=[
            pl.BlockSpec((ITEM_DIM, VC), lambda v: (0, v)),
            pl.BlockSpec((ITEM_DIM, CROSS), lambda v: (0, 0)),
        ],
        out_specs=pl.BlockSpec((VC, CROSS), lambda v: (v, 0)),
        out_shape=jax.ShapeDtypeStruct((ITEM_VOCAB, CROSS), jnp.float32),
        compiler_params=pltpu.CompilerParams(
            dimension_semantics=("parallel",),
        ),
    )(item_t, wdev)


@functools.lru_cache(maxsize=1)
def _sc_gather_build():
    mesh = plsc.VectorSubcoreMesh(core_axis_name="c", subcore_axis_name="s")

    @functools.partial(
        pl.kernel,
        mesh=mesh,
        out_type=(
            jax.ShapeDtypeStruct((N_DEEP, B, CROSS), jnp.float32),
            jax.ShapeDtypeStruct((B, CROSS), jnp.float32),
        ),
        scratch_types=[
            pltpu.VMEM((BPW,), jnp.int32),
            pltpu.VMEM((BPW,), jnp.int32),
            pltpu.VMEM((BPW, CROSS), jnp.float32),
            pltpu.VMEM((BPW, CROSS), jnp.float32),
            pltpu.SemaphoreType.DMA,
        ],
    )
    def sc_gather(
        p_deep_hbm,
        deep_idx_hbm,
        p_item_hbm,
        dev_idx_hbm,
        deep_out_hbm,
        dev_out_hbm,
        dev_idx_v,
        cur_idx_v,
        rows_v,
        item_rows_v,
        gsem,
    ):
        wid = lax.axis_index("s") * NUM_CORES + lax.axis_index("c")
        base = wid * BPW

        pltpu.sync_copy(dev_idx_hbm.at[pl.ds(base, BPW)], dev_idx_v)
        pltpu.async_copy(p_item_hbm.at[dev_idx_v], item_rows_v, gsem).wait()
        pltpu.sync_copy(item_rows_v, dev_out_hbm.at[pl.ds(base, BPW)])

        def body(t, _):
            pltpu.sync_copy(deep_idx_hbm.at[pl.ds(t * B + base, BPW)], cur_idx_v)
            pltpu.async_copy(p_deep_hbm.at[cur_idx_v], rows_v, gsem).wait()
            pltpu.sync_copy(rows_v, deep_out_hbm.at[t, pl.ds(base, BPW)])
            return _

        lax.fori_loop(0, N_DEEP, body, None)

    return sc_gather


def _combine_body(g_ref, dev_ref, wide_ref, wwide_ref, b_ref, out_ref):
    acc = jnp.sum(g_ref[...], axis=0) + dev_ref[...]
    wblk = wide_ref[...]
    mean = jnp.mean(wblk, axis=0, keepdims=True)
    var = jnp.mean(jnp.square(wblk - mean), axis=0, keepdims=True)
    wn = (wblk - mean) * lax.rsqrt(var + 1e-5)
    wide_part = lax.dot_general(
        wn, wwide_ref[...], (((0,), (0,)), ((), ())),
        preferred_element_type=jnp.float32,
        precision=lax.Precision.HIGHEST,
    )
    out_ref[...] = acc + wide_part + b_ref[...]


def _combine(gathered, dev, wide_in, wwide, b2):
    BT = 512
    return pl.pallas_call(
        _combine_body,
        grid=(B // BT,),
        in_specs=[
            pl.BlockSpec((N_DEEP, BT, CROSS), lambda bb: (0, bb, 0)),
            pl.BlockSpec((BT, CROSS), lambda bb: (bb, 0)),
            pl.BlockSpec((NUM_WIDE, BT), lambda bb: (0, bb)),
            pl.BlockSpec((NUM_WIDE, CROSS), lambda bb: (0, 0)),
            pl.BlockSpec((1, CROSS), lambda bb: (0, 0)),
        ],
        out_specs=pl.BlockSpec((BT, CROSS), lambda bb: (bb, 0)),
        out_shape=jax.ShapeDtypeStruct((B, CROSS), jnp.float32),
        compiler_params=pltpu.CompilerParams(
            dimension_semantics=("parallel",),
        ),
    )(gathered, dev, wide_in, wwide, b2)


def kernel(deep_in, wide_in, device_in, deep_tables, item_table, ln_gamma, ln_beta, W, b):
    deep_in = deep_in.astype(jnp.int32)
    deep_t = jnp.transpose(deep_tables, (0, 2, 1))
    item_t = jnp.transpose(item_table)

    wd = W[: N_DEEP * DEEP_DIM].reshape(N_DEEP, DEEP_DIM, CROSS)
    wdev = W[N_DEEP * DEEP_DIM : N_DEEP * DEEP_DIM + ITEM_DIM]
    w_wide_raw = W[N_DEEP * DEEP_DIM + ITEM_DIM :]
    wwide = ln_gamma[:, None] * w_wide_raw
    b2 = (b + ln_beta @ w_wide_raw).reshape(1, CROSS)

    p_deep = _project_deep(deep_t, wd).reshape(N_DEEP * DEEP_VOCAB, CROSS)
    p_item = _project_item(item_t, wdev)

    offs = (jnp.arange(N_DEEP, dtype=jnp.int32) * DEEP_VOCAB)[:, None]
    flat_idx = (deep_in + offs).reshape(N_DEEP * B)
    gathered, dev = _sc_gather_build()(
        p_deep, flat_idx, p_item, device_in.astype(jnp.int32)
    )

    return _combine(gathered, dev, wide_in, wwide, b2)


# speedup vs baseline: 6.1259x; 1.0481x over previous
import functools

import jax
import jax.numpy as jnp
from jax import lax
from jax.experimental import pallas as pl
from jax.experimental.pallas import tpu as pltpu
from jax.experimental.pallas import tpu_sc as plsc

B = 4096
N_DEEP = 26
DEEP_VOCAB = 100000
DEEP_DIM = 100
ITEM_VOCAB = 1000000
ITEM_DIM = 64
NUM_WIDE = 26
CROSS = 128

NUM_CORES = 2
NUM_SUBCORES = 16
NW = NUM_CORES * NUM_SUBCORES
BPW = B // NW

VC = 16384


def _project_deep_body(t_ref, w_ref, p_ref):
    tb = t_ref[0].astype(jnp.bfloat16)
    w = w_ref[0].astype(jnp.bfloat16)
    p_ref[0] = lax.dot_general(
        tb, w, (((0,), (0,)), ((), ())),
        preferred_element_type=jnp.float32,
    )


def _project_deep(deep_t, wd):
    nvb = (DEEP_VOCAB + VC - 1) // VC
    return pl.pallas_call(
        _project_deep_body,
        grid=(N_DEEP, nvb),
        in_specs=[
            pl.BlockSpec((1, DEEP_DIM, VC), lambda i, v: (i, 0, v)),
            pl.BlockSpec((1, DEEP_DIM, CROSS), lambda i, v: (i, 0, 0)),
        ],
        out_specs=pl.BlockSpec((1, VC, CROSS), lambda i, v: (i, v, 0)),
        out_shape=jax.ShapeDtypeStruct((N_DEEP, DEEP_VOCAB, CROSS), jnp.float32),
        compiler_params=pltpu.CompilerParams(
            dimension_semantics=("parallel", "parallel"),
        ),
    )(deep_t, wd)


def _project_item_body(t_ref, w_ref, p_ref):
    tb = t_ref[...].astype(jnp.bfloat16)
    w = w_ref[...].astype(jnp.bfloat16)
    p_ref[...] = lax.dot_general(
        tb, w, (((0,), (0,)), ((), ())),
        preferred_element_type=jnp.float32,
    )


def _project_item(item_t, wdev):
    nvb = (ITEM_VOCAB + VC - 1) // VC
    return pl.pallas_call(
        _project_item_body,
        grid=(nvb,),
        in_specs=[
            pl.BlockSpec((ITEM_DIM, VC), lambda v: (0, v)),
            pl.BlockSpec((ITEM_DIM, CROSS), lambda v: (0, 0)),
        ],
        out_specs=pl.BlockSpec((VC, CROSS), lambda v: (v, 0)),
        out_shape=jax.ShapeDtypeStruct((ITEM_VOCAB, CROSS), jnp.float32),
        compiler_params=pltpu.CompilerParams(
            dimension_semantics=("parallel",),
        ),
    )(item_t, wdev)


@functools.lru_cache(maxsize=1)
def _sc_gather_build():
    mesh = plsc.VectorSubcoreMesh(core_axis_name="c", subcore_axis_name="s")

    @functools.partial(
        pl.kernel,
        mesh=mesh,
        out_type=(
            jax.ShapeDtypeStruct((N_DEEP, B, CROSS), jnp.float32),
            jax.ShapeDtypeStruct((B, CROSS), jnp.float32),
        ),
        scratch_types=[
            pltpu.VMEM((BPW,), jnp.int32),
            pltpu.VMEM((BPW,), jnp.int32),
            pltpu.VMEM((BPW, CROSS), jnp.float32),
            pltpu.VMEM((BPW, CROSS), jnp.float32),
            pltpu.SemaphoreType.DMA,
        ],
    )
    def sc_gather(
        p_deep_hbm,
        deep_idx_hbm,
        p_item_hbm,
        dev_idx_hbm,
        deep_out_hbm,
        dev_out_hbm,
        dev_idx_v,
        cur_idx_v,
        rows_v,
        item_rows_v,
        gsem,
    ):
        wid = lax.axis_index("s") * NUM_CORES + lax.axis_index("c")
        base = wid * BPW

        pltpu.sync_copy(dev_idx_hbm.at[pl.ds(base, BPW)], dev_idx_v)
        pltpu.async_copy(p_item_hbm.at[dev_idx_v], item_rows_v, gsem).wait()
        pltpu.sync_copy(item_rows_v, dev_out_hbm.at[pl.ds(base, BPW)])

        def body(t, _):
            pltpu.sync_copy(deep_idx_hbm.at[pl.ds(t * B + base, BPW)], cur_idx_v)
            pltpu.async_copy(p_deep_hbm.at[cur_idx_v], rows_v, gsem).wait()
            pltpu.sync_copy(rows_v, deep_out_hbm.at[t, pl.ds(base, BPW)])
            return _

        lax.fori_loop(0, N_DEEP, body, None)

    return sc_gather


def _combine_body(g_ref, dev_ref, wide_ref, wwide_ref, b_ref, out_ref):
    acc = jnp.sum(g_ref[...], axis=0) + dev_ref[...]
    wblk = wide_ref[...]
    mean = jnp.mean(wblk, axis=0, keepdims=True)
    var = jnp.mean(jnp.square(wblk - mean), axis=0, keepdims=True)
    wn = (wblk - mean) * lax.rsqrt(var + 1e-5)
    wide_part = lax.dot_general(
        wn, wwide_ref[...], (((0,), (0,)), ((), ())),
        preferred_element_type=jnp.float32,
        precision=lax.Precision.HIGHEST,
    )
    out_ref[...] = acc + wide_part + b_ref[...]


def _combine(gathered, dev, wide_in, wwide, b2):
    BT = 512
    return pl.pallas_call(
        _combine_body,
        grid=(B // BT,),
        in_specs=[
            pl.BlockSpec((N_DEEP, BT, CROSS), lambda bb: (0, bb, 0)),
            pl.BlockSpec((BT, CROSS), lambda bb: (bb, 0)),
            pl.BlockSpec((NUM_WIDE, BT), lambda bb: (0, bb)),
            pl.BlockSpec((NUM_WIDE, CROSS), lambda bb: (0, 0)),
            pl.BlockSpec((1, CROSS), lambda bb: (0, 0)),
        ],
        out_specs=pl.BlockSpec((BT, CROSS), lambda bb: (bb, 0)),
        out_shape=jax.ShapeDtypeStruct((B, CROSS), jnp.float32),
        compiler_params=pltpu.CompilerParams(
            dimension_semantics=("parallel",),
        ),
    )(gathered, dev, wide_in, wwide, b2)


def kernel(deep_in, wide_in, device_in, deep_tables, item_table, ln_gamma, ln_beta, W, b):
    deep_in = deep_in.astype(jnp.int32)
    deep_t = jnp.transpose(deep_tables, (0, 2, 1))
    item_t = jnp.transpose(item_table)

    wd = W[: N_DEEP * DEEP_DIM].reshape(N_DEEP, DEEP_DIM, CROSS)
    wdev = W[N_DEEP * DEEP_DIM : N_DEEP * DEEP_DIM + ITEM_DIM]
    w_wide_raw = W[N_DEEP * DEEP_DIM + ITEM_DIM :]
    wwide = ln_gamma[:, None] * w_wide_raw
    b2 = (b + ln_beta @ w_wide_raw).reshape(1, CROSS)

    p_deep = _project_deep(deep_t, wd).reshape(N_DEEP * DEEP_VOCAB, CROSS)
    p_item = _project_item(item_t, wdev)

    offs = (jnp.arange(N_DEEP, dtype=jnp.int32) * DEEP_VOCAB)[:, None]
    flat_idx = (deep_in + offs).reshape(N_DEEP * B)
    gathered, dev = _sc_gather_build()(
        p_deep, flat_idx, p_item, device_in.astype(jnp.int32)
    )

    return _combine(gathered, dev, wide_in, wwide, b2)


# speedup vs baseline: 6.1403x; 1.0024x over previous
import functools

import jax
import jax.numpy as jnp
from jax import lax
from jax.experimental import pallas as pl
from jax.experimental.pallas import tpu as pltpu
from jax.experimental.pallas import tpu_sc as plsc

B = 4096
N_DEEP = 26
DEEP_VOCAB = 100000
DEEP_DIM = 100
ITEM_VOCAB = 1000000
ITEM_DIM = 64
NUM_WIDE = 26
CROSS = 128

NUM_CORES = 2
NUM_SUBCORES = 16
NW = NUM_CORES * NUM_SUBCORES
BPW = B // NW

VC = 12800


def _project_deep_body(t_ref, w_ref, p_ref):
    tb = t_ref[0].astype(jnp.bfloat16)
    w = w_ref[0].astype(jnp.bfloat16)
    p_ref[0] = lax.dot_general(
        tb, w, (((0,), (0,)), ((), ())),
        preferred_element_type=jnp.float32,
    )


def _project_deep(deep_t, wd):
    nvb = (DEEP_VOCAB + VC - 1) // VC
    return pl.pallas_call(
        _project_deep_body,
        grid=(N_DEEP, nvb),
        in_specs=[
            pl.BlockSpec((1, DEEP_DIM, VC), lambda i, v: (i, 0, v)),
            pl.BlockSpec((1, DEEP_DIM, CROSS), lambda i, v: (i, 0, 0)),
        ],
        out_specs=pl.BlockSpec((1, VC, CROSS), lambda i, v: (i, v, 0)),
        out_shape=jax.ShapeDtypeStruct((N_DEEP, DEEP_VOCAB, CROSS), jnp.float32),
        compiler_params=pltpu.CompilerParams(
            dimension_semantics=("parallel", "parallel"),
        ),
    )(deep_t, wd)


def _project_item_body(t_ref, w_ref, p_ref):
    tb = t_ref[...].astype(jnp.bfloat16)
    w = w_ref[...].astype(jnp.bfloat16)
    p_ref[...] = lax.dot_general(
        tb, w, (((0,), (0,)), ((), ())),
        preferred_element_type=jnp.float32,
    )


def _project_item(item_t, wdev):
    nvb = (ITEM_VOCAB + VC - 1) // VC
    return pl.pallas_call(
        _project_item_body,
        grid=(nvb,),
        in_specs=[
            pl.BlockSpec((ITEM_DIM, VC), lambda v: (0, v)),
            pl.BlockSpec((ITEM_DIM, CROSS), lambda v: (0, 0)),
        ],
        out_specs=pl.BlockSpec((VC, CROSS), lambda v: (v, 0)),
        out_shape=jax.ShapeDtypeStruct((ITEM_VOCAB, CROSS), jnp.float32),
        compiler_params=pltpu.CompilerParams(
            dimension_semantics=("parallel",),
        ),
    )(item_t, wdev)


@functools.lru_cache(maxsize=1)
def _sc_gather_build():
    mesh = plsc.VectorSubcoreMesh(core_axis_name="c", subcore_axis_name="s")

    @functools.partial(
        pl.kernel,
        mesh=mesh,
        out_type=(
            jax.ShapeDtypeStruct((N_DEEP, B, CROSS), jnp.float32),
            jax.ShapeDtypeStruct((B, CROSS), jnp.float32),
        ),
        scratch_types=[
            pltpu.VMEM((BPW,), jnp.int32),
            pltpu.VMEM((BPW,), jnp.int32),
            pltpu.VMEM((BPW, CROSS), jnp.float32),
            pltpu.VMEM((BPW, CROSS), jnp.float32),
            pltpu.SemaphoreType.DMA,
        ],
    )
    def sc_gather(
        p_deep_hbm,
        deep_idx_hbm,
        p_item_hbm,
        dev_idx_hbm,
        deep_out_hbm,
        dev_out_hbm,
        dev_idx_v,
        cur_idx_v,
        rows_v,
        item_rows_v,
        gsem,
    ):
        wid = lax.axis_index("s") * NUM_CORES + lax.axis_index("c")
        base = wid * BPW

        pltpu.sync_copy(dev_idx_hbm.at[pl.ds(base, BPW)], dev_idx_v)
        pltpu.async_copy(p_item_hbm.at[dev_idx_v], item_rows_v, gsem).wait()
        pltpu.sync_copy(item_rows_v, dev_out_hbm.at[pl.ds(base, BPW)])

        def body(t, _):
            pltpu.sync_copy(deep_idx_hbm.at[pl.ds(t * B + base, BPW)], cur_idx_v)
            pltpu.async_copy(p_deep_hbm.at[cur_idx_v], rows_v, gsem).wait()
            pltpu.sync_copy(rows_v, deep_out_hbm.at[t, pl.ds(base, BPW)])
            return _

        lax.fori_loop(0, N_DEEP, body, None)

    return sc_gather


def _combine_body(g_ref, dev_ref, wide_ref, wwide_ref, b_ref, out_ref):
    acc = jnp.sum(g_ref[...], axis=0) + dev_ref[...]
    wblk = wide_ref[...]
    mean = jnp.mean(wblk, axis=0, keepdims=True)
    var = jnp.mean(jnp.square(wblk - mean), axis=0, keepdims=True)
    wn = (wblk - mean) * lax.rsqrt(var + 1e-5)
    wide_part = lax.dot_general(
        wn, wwide_ref[...], (((0,), (0,)), ((), ())),
        preferred_element_type=jnp.float32,
        precision=lax.Precision.HIGHEST,
    )
    out_ref[...] = acc + wide_part + b_ref[...]


def _combine(gathered, dev, wide_in, wwide, b2):
    BT = 512
    return pl.pallas_call(
        _combine_body,
        grid=(B // BT,),
        in_specs=[
            pl.BlockSpec((N_DEEP, BT, CROSS), lambda bb: (0, bb, 0)),
            pl.BlockSpec((BT, CROSS), lambda bb: (bb, 0)),
            pl.BlockSpec((NUM_WIDE, BT), lambda bb: (0, bb)),
            pl.BlockSpec((NUM_WIDE, CROSS), lambda bb: (0, 0)),
            pl.BlockSpec((1, CROSS), lambda bb: (0, 0)),
        ],
        out_specs=pl.BlockSpec((BT, CROSS), lambda bb: (bb, 0)),
        out_shape=jax.ShapeDtypeStruct((B, CROSS), jnp.float32),
        compiler_params=pltpu.CompilerParams(
            dimension_semantics=("parallel",),
        ),
    )(gathered, dev, wide_in, wwide, b2)


def kernel(deep_in, wide_in, device_in, deep_tables, item_table, ln_gamma, ln_beta, W, b):
    deep_in = deep_in.astype(jnp.int32)
    deep_t = jnp.transpose(deep_tables, (0, 2, 1))
    item_t = jnp.transpose(item_table)

    wd = W[: N_DEEP * DEEP_DIM].reshape(N_DEEP, DEEP_DIM, CROSS)
    wdev = W[N_DEEP * DEEP_DIM : N_DEEP * DEEP_DIM + ITEM_DIM]
    w_wide_raw = W[N_DEEP * DEEP_DIM + ITEM_DIM :]
    wwide = ln_gamma[:, None] * w_wide_raw
    b2 = (b + ln_beta @ w_wide_raw).reshape(1, CROSS)

    p_deep = _project_deep(deep_t, wd).reshape(N_DEEP * DEEP_VOCAB, CROSS)
    p_item = _project_item(item_t, wdev)

    offs = (jnp.arange(N_DEEP, dtype=jnp.int32) * DEEP_VOCAB)[:, None]
    flat_idx = (deep_in + offs).reshape(N_DEEP * B)
    gathered, dev = _sc_gather_build()(
        p_deep, flat_idx, p_item, device_in.astype(jnp.int32)
    )

    return _combine(gathered, dev, wide_in, wwide, b2)
